# Initial kernel scaffold; baseline (speedup 1.0000x reference)
import jax
import jax.numpy as jnp
from jax.experimental import pallas as pl


def kernel(flat, cu_seqlens):
    raise NotImplementedError("write your pallas kernel here")


# speedup vs baseline: 1.8253x; 1.8253x over previous
import functools

import jax
import jax.numpy as jnp
from jax import lax
from jax.experimental import pallas as pl
from jax.experimental.pallas import tpu as pltpu
from jax.experimental.pallas import tpu_sc as plsc

B = 16
S = 2048
T = 16384
D = 512

NW = 32
RPW = B * S // NW
CH = 64
NCH = RPW // CH

_mesh = plsc.VectorSubcoreMesh(core_axis_name="c", subcore_axis_name="s")


@functools.partial(
    pl.kernel,
    mesh=_mesh,
    out_type=jax.ShapeDtypeStruct((B, S, D), jnp.float32),
    scratch_types=[
        pltpu.VMEM((32,), jnp.int32),
        pltpu.VMEM((CH,), jnp.int32),
        pltpu.VMEM((CH, D), jnp.float32),
        pltpu.VMEM((CH, D), jnp.float32),
        pltpu.SemaphoreType.DMA,
    ],
)
def _sc_unflatten(flat_hbm, cu_hbm, out_hbm, cu_v, idx_v, buf, zbuf, sem):
    wid = lax.axis_index("c") * 16 + lax.axis_index("s")
    b = wid // 2
    s_base = (wid % 2) * (S // 2)

    def _zero(i, carry):
        r = i // (D // 16)
        c = i % (D // 16)
        zbuf[r, pl.ds(c * 16, 16)] = jnp.zeros((16,), jnp.float32)
        return carry

    lax.fori_loop(0, CH * D // 16, _zero, 0)

    pltpu.sync_copy(cu_hbm, cu_v)
    lane = lax.broadcasted_iota(jnp.int32, (16,), 0)
    cu_b = cu_v[pl.ds(b, 16)]
    start = cu_b[0]
    end = cu_b[1]
    nvalid = jnp.clip(end - start - s_base, 0, RPW)

    for c in range(NCH):
        lo = c * CH
        dst = out_hbm.at[b, pl.ds(s_base + lo, CH)]
        empty = nvalid <= lo

        @pl.when(empty)
        def _():
            pltpu.sync_copy(zbuf, dst)

        @pl.when(~empty)
        def _():
            base = start + s_base + lo
            for j in range(CH // 16):
                idx_v[pl.ds(j * 16, 16)] = jnp.minimum(
                    base + j * 16 + lane, end - 1
                )
            pltpu.async_copy(flat_hbm.at[idx_v], buf, sem).wait()
            nv = nvalid - lo

            def _ztail(i, carry):
                r = nv + i // (D // 16)
                c2 = i % (D // 16)
                buf[r, pl.ds(c2 * 16, 16)] = jnp.zeros((16,), jnp.float32)
                return carry

            lax.fori_loop(0, (CH - jnp.minimum(nv, CH)) * (D // 16), _ztail, 0)
            pltpu.sync_copy(buf, dst)


def _mask_body(cu_ref, mask_ref):
    col = lax.broadcasted_iota(jnp.int32, (1, S), 1)
    for b in range(B):
        ln = cu_ref[b + 1] - cu_ref[b]
        mask_ref[pl.ds(b, 1), :] = col < ln


_mask_call = pl.pallas_call(
    _mask_body,
    in_specs=[pl.BlockSpec(memory_space=pltpu.SMEM)],
    out_specs=pl.BlockSpec(memory_space=pltpu.VMEM),
    out_shape=jax.ShapeDtypeStruct((B, S), jnp.bool_),
)


def kernel(flat, cu_seqlens):
    cu_p = jnp.pad(cu_seqlens.astype(jnp.int32), (0, 32 - (B + 1)))
    out = _sc_unflatten(flat, cu_p)
    mask = _mask_call(cu_p)
    return out, mask


# speedup vs baseline: 2.1064x; 1.1540x over previous
import functools

import jax
import jax.numpy as jnp
from jax import lax
from jax.experimental import pallas as pl
from jax.experimental.pallas import tpu as pltpu
from jax.experimental.pallas import tpu_sc as plsc

B = 16
S = 2048
T = 16384
D = 512

NW = 32
RPW = B * S // NW
CH = 64
NCH = RPW // CH

_mesh = plsc.VectorSubcoreMesh(core_axis_name="c", subcore_axis_name="s")


@functools.partial(
    pl.kernel,
    mesh=_mesh,
    out_type=jax.ShapeDtypeStruct((B, S, D), jnp.float32),
    scratch_types=[
        pltpu.VMEM((32,), jnp.int32),
        pltpu.VMEM((CH,), jnp.int32),
        pltpu.VMEM((CH,), jnp.int32),
        pltpu.VMEM((CH, D), jnp.float32),
        pltpu.VMEM((CH, D), jnp.float32),
        pltpu.VMEM((CH, D), jnp.float32),
        pltpu.SemaphoreType.DMA,
        pltpu.SemaphoreType.DMA,
        pltpu.SemaphoreType.DMA,
    ],
)
def _sc_unflatten(
    flat_hbm, cu_hbm, out_hbm, cu_v, idx0, idx1, buf0, buf1, zbuf,
    sem_g, sem_w, sem_z,
):
    idx = [idx0, idx1]
    buf = [buf0, buf1]
    wid = lax.axis_index("c") * 16 + lax.axis_index("s")
    b = wid // 2
    s_base = (wid % 2) * (S // 2)

    def _zero(i, carry):
        r = i // 4
        col = (i % 4) * 128
        for u in range(8):
            zbuf[r, pl.ds(col + u * 16, 16)] = jnp.zeros((16,), jnp.float32)
        return carry

    lax.fori_loop(0, CH * D // 128, _zero, 0)

    pltpu.sync_copy(cu_hbm, cu_v)
    lane = lax.broadcasted_iota(jnp.int32, (16,), 0)
    cu_b = cu_v[pl.ds(b, 16)]
    start = cu_b[0]
    end = cu_b[1]
    nvalid = jnp.clip(end - start - s_base, 0, RPW)
    nch_v = (nvalid + CH - 1) // CH

    for c in range(NCH):
        dst = out_hbm.at[b, pl.ds(s_base + c * CH, CH)]

        @pl.when(c >= nch_v)
        def _():
            pltpu.async_copy(zbuf, dst, sem_z)

        @pl.when(c < nch_v)
        def _():
            if c >= 2:
                prev = out_hbm.at[b, pl.ds(s_base + (c - 2) * CH, CH)]
                pltpu.make_async_copy(buf[c & 1], prev, sem_w).wait()
            base = start + s_base + c * CH
            for j in range(CH // 16):
                idx[c & 1][pl.ds(j * 16, 16)] = jnp.minimum(
                    base + j * 16 + lane, end - 1
                )
            pltpu.async_copy(flat_hbm.at[idx[c & 1]], buf[c & 1], sem_g).wait()
            nv = nvalid - c * CH
            def _ztail(i, carry):
                r = nv + i // (D // 16)
                col = (i % (D // 16)) * 16
                buf[c & 1][r, pl.ds(col, 16)] = jnp.zeros((16,), jnp.float32)
                return carry

            lax.fori_loop(
                0, (CH - jnp.minimum(nv, CH)) * (D // 16), _ztail, 0
            )
            pltpu.async_copy(buf[c & 1], dst, sem_w)

    def _drain_w(i, carry):
        pltpu.make_async_copy(
            buf0, out_hbm.at[b, pl.ds(s_base, CH)], sem_w
        ).wait()
        return carry

    lax.fori_loop(0, jnp.minimum(nch_v, 2), _drain_w, 0)

    def _drain_z(i, carry):
        pltpu.make_async_copy(
            zbuf, out_hbm.at[b, pl.ds(s_base, CH)], sem_z
        ).wait()
        return carry

    lax.fori_loop(0, NCH - nch_v, _drain_z, 0)


def _mask_body(cu_ref, mask_ref):
    col = lax.broadcasted_iota(jnp.int32, (1, S), 1)
    for b in range(B):
        ln = cu_ref[b + 1] - cu_ref[b]
        mask_ref[pl.ds(b, 1), :] = col < ln


_mask_call = pl.pallas_call(
    _mask_body,
    in_specs=[pl.BlockSpec(memory_space=pltpu.SMEM)],
    out_specs=pl.BlockSpec(memory_space=pltpu.VMEM),
    out_shape=jax.ShapeDtypeStruct((B, S), jnp.bool_),
)


def kernel(flat, cu_seqlens):
    cu_p = jnp.pad(cu_seqlens.astype(jnp.int32), (0, 32 - (B + 1)))
    out = _sc_unflatten(flat, cu_p)
    mask = _mask_call(cu_p)
    return out, mask


# speedup vs baseline: 2.3492x; 1.1153x over previous
import functools

import jax
import jax.numpy as jnp
from jax import lax
from jax.experimental import pallas as pl
from jax.experimental.pallas import tpu as pltpu
from jax.experimental.pallas import tpu_sc as plsc

B = 16
S = 2048
T = 16384
D = 512

NW = 32
RPW = B * S // NW
CH = 64
NCH = RPW // CH
NBUF = 3
ZR = 32

_mesh = plsc.VectorSubcoreMesh(core_axis_name="c", subcore_axis_name="s")


@functools.partial(
    pl.kernel,
    mesh=_mesh,
    out_type=jax.ShapeDtypeStruct((B, S, D), jnp.float32),
    scratch_types=[
        pltpu.VMEM((32,), jnp.int32),
        pltpu.VMEM((CH,), jnp.int32),
        pltpu.VMEM((CH,), jnp.int32),
        pltpu.VMEM((CH,), jnp.int32),
        pltpu.VMEM((CH, D), jnp.float32),
        pltpu.VMEM((CH, D), jnp.float32),
        pltpu.VMEM((CH, D), jnp.float32),
        pltpu.VMEM((ZR, D), jnp.float32),
        pltpu.SemaphoreType.DMA,
        pltpu.SemaphoreType.DMA,
        pltpu.SemaphoreType.DMA,
        pltpu.SemaphoreType.DMA,
        pltpu.SemaphoreType.DMA,
        pltpu.SemaphoreType.DMA,
        pltpu.SemaphoreType.DMA,
    ],
)
def _sc_unflatten(
    flat_hbm, cu_hbm, out_hbm, cu_v, idx0, idx1, idx2, buf0, buf1, buf2,
    zbuf, sg0, sg1, sg2, sw0, sw1, sw2, sem_z,
):
    idx = [idx0, idx1, idx2]
    buf = [buf0, buf1, buf2]
    sem_g = [sg0, sg1, sg2]
    sem_w = [sw0, sw1, sw2]
    wid = lax.axis_index("c") * 16 + lax.axis_index("s")
    b = wid // 2
    s_base = (wid % 2) * (S // 2)

    def _zero(i, carry):
        r = i // 4
        col = (i % 4) * 128
        for u in range(8):
            zbuf[r, pl.ds(col + u * 16, 16)] = jnp.zeros((16,), jnp.float32)
        return carry

    lax.fori_loop(0, ZR * D // 128, _zero, 0)

    pltpu.sync_copy(cu_hbm, cu_v)
    lane = lax.broadcasted_iota(jnp.int32, (16,), 0)
    cu_b = cu_v[pl.ds(b, 16)]
    start = cu_b[0]
    end = cu_b[1]
    nvalid = jnp.clip(end - start - s_base, 0, RPW)
    nch_v = (nvalid + CH - 1) // CH

    def _dst(c):
        return out_hbm.at[b, pl.ds(s_base + c * CH, CH)]

    for c in range(NCH + 1):
        if c < NCH:
            k = c % NBUF

            @pl.when(c < nch_v)
            def _():
                if c >= NBUF:
                    pltpu.make_async_copy(
                        buf[k], _dst(c - NBUF), sem_w[k]
                    ).wait()
                base = start + s_base + c * CH
                for j in range(CH // 16):
                    idx[k][pl.ds(j * 16, 16)] = jnp.minimum(
                        base + j * 16 + lane, end - 1
                    )
                pltpu.async_copy(flat_hbm.at[idx[k]], buf[k], sem_g[k])

            @pl.when(c >= nch_v)
            def _():
                pltpu.async_copy(
                    zbuf, out_hbm.at[b, pl.ds(s_base + c * CH, ZR)], sem_z
                )
                pltpu.async_copy(
                    zbuf,
                    out_hbm.at[b, pl.ds(s_base + c * CH + ZR, ZR)],
                    sem_z,
                )

        cp = c - 1
        if cp >= 0:
            kp = cp % NBUF

            @pl.when(cp < nch_v)
            def _():
                pltpu.make_async_copy(buf[kp], _dst(cp), sem_g[kp]).wait()
                nv = nvalid - cp * CH
                def _ztail(i, carry):
                    r = nv + i // (D // 16)
                    col = (i % (D // 16)) * 16
                    buf[kp][r, pl.ds(col, 16)] = jnp.zeros(
                        (16,), jnp.float32
                    )
                    return carry

                lax.fori_loop(
                    0, (CH - jnp.minimum(nv, CH)) * (D // 16), _ztail, 0
                )
                pltpu.async_copy(buf[kp], _dst(cp), sem_w[kp])

    for k in range(NBUF):
        @pl.when(k < nch_v)
        def _():
            pltpu.make_async_copy(
                buf[k], out_hbm.at[b, pl.ds(s_base, CH)], sem_w[k]
            ).wait()

    def _drain_z(i, carry):
        pltpu.make_async_copy(
            zbuf, out_hbm.at[b, pl.ds(s_base, ZR)], sem_z
        ).wait()
        return carry

    lax.fori_loop(0, 2 * (NCH - nch_v), _drain_z, 0)


def _mask_body(cu_ref, mask_ref):
    col = lax.broadcasted_iota(jnp.int32, (1, S), 1)
    for b in range(B):
        ln = cu_ref[b + 1] - cu_ref[b]
        mask_ref[pl.ds(b, 1), :] = col < ln


_mask_call = pl.pallas_call(
    _mask_body,
    in_specs=[pl.BlockSpec(memory_space=pltpu.SMEM)],
    out_specs=pl.BlockSpec(memory_space=pltpu.VMEM),
    out_shape=jax.ShapeDtypeStruct((B, S), jnp.bool_),
)


def kernel(flat, cu_seqlens):
    cu_p = jnp.pad(cu_seqlens.astype(jnp.int32), (0, 32 - (B + 1)))
    out = _sc_unflatten(flat, cu_p)
    mask = _mask_call(cu_p)
    return out, mask


# speedup vs baseline: 2.5073x; 1.0673x over previous
import functools

import jax
import jax.numpy as jnp
from jax import lax
from jax.experimental import pallas as pl
from jax.experimental.pallas import tpu as pltpu
from jax.experimental.pallas import tpu_sc as plsc

B = 16
S = 2048
T = 16384
D = 512

NW = 32
RPW = B * S // NW
CH = 64
NCH = RPW // CH
NBUF = 3
ZR = 32

_mesh = plsc.VectorSubcoreMesh(core_axis_name="c", subcore_axis_name="s")


@functools.partial(
    pl.kernel,
    mesh=_mesh,
    out_type=jax.ShapeDtypeStruct((B, S, D), jnp.float32),
    scratch_types=[
        pltpu.VMEM((32,), jnp.int32),
        pltpu.VMEM((CH,), jnp.int32),
        pltpu.VMEM((CH,), jnp.int32),
        pltpu.VMEM((CH,), jnp.int32),
        pltpu.VMEM((CH, D), jnp.float32),
        pltpu.VMEM((CH, D), jnp.float32),
        pltpu.VMEM((CH, D), jnp.float32),
        pltpu.VMEM((ZR, D), jnp.float32),
        pltpu.SemaphoreType.DMA,
        pltpu.SemaphoreType.DMA,
        pltpu.SemaphoreType.DMA,
        pltpu.SemaphoreType.DMA,
        pltpu.SemaphoreType.DMA,
        pltpu.SemaphoreType.DMA,
        pltpu.SemaphoreType.DMA,
    ],
)
def _sc_unflatten(
    flat_hbm, cu_hbm, out_hbm, cu_v, idx0, idx1, idx2, buf0, buf1, buf2,
    zbuf, sg0, sg1, sg2, sw0, sw1, sw2, sem_z,
):
    idx = [idx0, idx1, idx2]
    buf = [buf0, buf1, buf2]
    sem_g = [sg0, sg1, sg2]
    sem_w = [sw0, sw1, sw2]
    wid = lax.axis_index("c") * 16 + lax.axis_index("s")

    cu_cp = pltpu.async_copy(cu_hbm, cu_v, sg0)

    def _zero(i, carry):
        r = i // 4
        col = (i % 4) * 128
        for u in range(8):
            zbuf[r, pl.ds(col + u * 16, 16)] = jnp.zeros((16,), jnp.float32)
        return carry

    lax.fori_loop(0, ZR * D // 128, _zero, 0)
    cu_cp.wait()

    lane = lax.broadcasted_iota(jnp.int32, (16,), 0)

    pos = []
    nv_s = []
    base_s = []
    end_s = []
    valid = []
    for t in range(B):
        p = (wid + 2 * t) % (S // CH)
        cu_b = cu_v[pl.ds(t, 16)]
        start = cu_b[0]
        end = cu_b[1]
        nv = jnp.clip(end - start - p * CH, 0, CH)
        pos.append(p)
        nv_s.append(nv)
        base_s.append(start + p * CH)
        end_s.append(end)
        valid.append(nv > 0)

    def _dst(t):
        return out_hbm.at[t, pl.ds(pos[t] * CH, CH)]

    for t in range(B + 1):
        if t < B:
            k = t % NBUF
            if t >= NBUF:
                @pl.when(valid[t - NBUF])
                def _():
                    pltpu.make_async_copy(
                        buf[k], _dst(t - NBUF), sem_w[k]
                    ).wait()

            @pl.when(valid[t])
            def _():
                for j in range(CH // 16):
                    idx[k][pl.ds(j * 16, 16)] = jnp.minimum(
                        base_s[t] + j * 16 + lane, end_s[t] - 1
                    )
                pltpu.async_copy(flat_hbm.at[idx[k]], buf[k], sem_g[k])

            @pl.when(jnp.logical_not(valid[t]))
            def _():
                pltpu.async_copy(
                    zbuf, out_hbm.at[t, pl.ds(pos[t] * CH, ZR)], sem_z
                )
                pltpu.async_copy(
                    zbuf, out_hbm.at[t, pl.ds(pos[t] * CH + ZR, ZR)], sem_z
                )

        tp = t - 1
        if tp >= 0:
            kp = tp % NBUF

            @pl.when(valid[tp])
            def _():
                pltpu.make_async_copy(buf[kp], _dst(tp), sem_g[kp]).wait()
                nv = nv_s[tp]
                def _ztail(i, carry):
                    r = nv + i // 4
                    col = (i % 4) * 128
                    for u in range(8):
                        buf[kp][r, pl.ds(col + u * 16, 16)] = jnp.zeros(
                            (16,), jnp.float32
                        )
                    return carry

                lax.fori_loop(0, (CH - nv) * (D // 128), _ztail, 0)
                pltpu.async_copy(buf[kp], _dst(tp), sem_w[kp])

    for t in range(B - NBUF, B):
        @pl.when(valid[t])
        def _():
            pltpu.make_async_copy(
                buf[t % NBUF], _dst(t), sem_w[t % NBUF]
            ).wait()

    n_zero = 0
    for t in range(B):
        n_zero = n_zero + jnp.where(valid[t], 0, 2)

    def _drain_z(i, carry):
        pltpu.make_async_copy(
            zbuf, out_hbm.at[0, pl.ds(0, ZR)], sem_z
        ).wait()
        return carry

    lax.fori_loop(0, n_zero, _drain_z, 0)


def _mask_body(cu_ref, mask_ref):
    col = lax.broadcasted_iota(jnp.int32, (1, S), 1)
    for b in range(B):
        ln = cu_ref[b + 1] - cu_ref[b]
        mask_ref[pl.ds(b, 1), :] = col < ln


_mask_call = pl.pallas_call(
    _mask_body,
    in_specs=[pl.BlockSpec(memory_space=pltpu.SMEM)],
    out_specs=pl.BlockSpec(memory_space=pltpu.VMEM),
    out_shape=jax.ShapeDtypeStruct((B, S), jnp.bool_),
)


def kernel(flat, cu_seqlens):
    cu_p = jnp.pad(cu_seqlens.astype(jnp.int32), (0, 32 - (B + 1)))
    out = _sc_unflatten(flat, cu_p)
    mask = _mask_call(cu_p)
    return out, mask


# speedup vs baseline: 2.8179x; 1.1239x over previous
import functools

import jax
import jax.numpy as jnp
from jax import lax
from jax.experimental import pallas as pl
from jax.experimental.pallas import tpu as pltpu
from jax.experimental.pallas import tpu_sc as plsc

B = 16
S = 2048
T = 16384
D = 512

NW = 32
CH = 32
NPOS = S // CH
NSLOT = 32
NBUF = 6
LAG = 2

_mesh = plsc.VectorSubcoreMesh(core_axis_name="c", subcore_axis_name="s")


@functools.partial(
    pl.kernel,
    mesh=_mesh,
    out_type=jax.ShapeDtypeStruct((B, S, D), jnp.float32),
    scratch_types=[
        pltpu.VMEM((32,), jnp.int32),
        pltpu.VMEM((NBUF, CH), jnp.int32),
        pltpu.VMEM((CH, D), jnp.float32),
        pltpu.VMEM((CH, D), jnp.float32),
        pltpu.VMEM((CH, D), jnp.float32),
        pltpu.VMEM((CH, D), jnp.float32),
        pltpu.VMEM((CH, D), jnp.float32),
        pltpu.VMEM((CH, D), jnp.float32),
        pltpu.VMEM((CH, D), jnp.float32),
        pltpu.SemaphoreType.DMA,
        pltpu.SemaphoreType.DMA,
        pltpu.SemaphoreType.DMA,
        pltpu.SemaphoreType.DMA,
        pltpu.SemaphoreType.DMA,
        pltpu.SemaphoreType.DMA,
        pltpu.SemaphoreType.DMA,
        pltpu.SemaphoreType.DMA,
        pltpu.SemaphoreType.DMA,
        pltpu.SemaphoreType.DMA,
        pltpu.SemaphoreType.DMA,
        pltpu.SemaphoreType.DMA,
        pltpu.SemaphoreType.DMA,
    ],
)
def _sc_unflatten(
    flat_hbm, cu_hbm, out_hbm, cu_v, idx_v,
    buf0, buf1, buf2, buf3, buf4, buf5, zbuf,
    sg0, sg1, sg2, sg3, sg4, sg5,
    sw0, sw1, sw2, sw3, sw4, sw5, sem_z,
):
    buf = [buf0, buf1, buf2, buf3, buf4, buf5]
    sem_g = [sg0, sg1, sg2, sg3, sg4, sg5]
    sem_w = [sw0, sw1, sw2, sw3, sw4, sw5]
    wid = lax.axis_index("c") * 16 + lax.axis_index("s")

    cu_cp = pltpu.async_copy(cu_hbm, cu_v, sg0)

    def _zero(i, carry):
        r = i // 4
        col = (i % 4) * 128
        for u in range(8):
            zbuf[r, pl.ds(col + u * 16, 16)] = jnp.zeros((16,), jnp.float32)
        return carry

    lax.fori_loop(0, CH * D // 128, _zero, 0)
    cu_cp.wait()

    lane = lax.broadcasted_iota(jnp.int32, (16,), 0)

    start_e = []
    end_e = []
    for t in range(B):
        cu_b = cu_v[pl.ds(t, 16)]
        start_e.append(cu_b[0])
        end_e.append(cu_b[1])

    pos = []
    nv_s = []
    base_s = []
    end_s = []
    valid = []
    for s in range(NSLOT):
        t = s % B
        p = (wid + 2 * t + CH * (s // B)) % NPOS
        nv = jnp.clip(end_e[t] - start_e[t] - p * CH, 0, CH)
        pos.append(p)
        nv_s.append(nv)
        base_s.append(start_e[t] + p * CH)
        end_s.append(end_e[t])
        valid.append(nv > 0)

    def _dst(s):
        return out_hbm.at[s % B, pl.ds(pos[s] * CH, CH)]

    for s in range(NSLOT + LAG):
        if s < NSLOT:
            k = s % NBUF
            if s >= NBUF:
                @pl.when(valid[s - NBUF])
                def _():
                    pltpu.make_async_copy(
                        buf[k], _dst(s - NBUF), sem_w[k]
                    ).wait()

            @pl.when(valid[s])
            def _():
                for j in range(CH // 16):
                    idx_v[k, pl.ds(j * 16, 16)] = jnp.minimum(
                        base_s[s] + j * 16 + lane, end_s[s] - 1
                    )
                pltpu.async_copy(
                    flat_hbm.at[idx_v.at[k]], buf[k], sem_g[k]
                )

            @pl.when(jnp.logical_not(valid[s]))
            def _():
                pltpu.async_copy(zbuf, _dst(s), sem_z)

        sp = s - LAG
        if sp >= 0:
            kp = sp % NBUF

            @pl.when(valid[sp])
            def _():
                pltpu.make_async_copy(buf[kp], _dst(sp), sem_g[kp]).wait()
                nv = nv_s[sp]
                def _ztail(i, carry):
                    r = nv + i // 4
                    col = (i % 4) * 128
                    for u in range(8):
                        buf[kp][r, pl.ds(col + u * 16, 16)] = jnp.zeros(
                            (16,), jnp.float32
                        )
                    return carry

                lax.fori_loop(0, (CH - nv) * (D // 128), _ztail, 0)
                pltpu.async_copy(buf[kp], _dst(sp), sem_w[kp])

    for s in range(NSLOT - NBUF, NSLOT):
        @pl.when(valid[s])
        def _():
            pltpu.make_async_copy(
                buf[s % NBUF], _dst(s), sem_w[s % NBUF]
            ).wait()

    n_zero = 0
    for s in range(NSLOT):
        n_zero = n_zero + jnp.where(valid[s], 0, 1)

    def _drain_z(i, carry):
        pltpu.make_async_copy(
            zbuf, out_hbm.at[0, pl.ds(0, CH)], sem_z
        ).wait()
        return carry

    lax.fori_loop(0, n_zero, _drain_z, 0)


def _mask_body(cu_ref, mask_ref):
    col = lax.broadcasted_iota(jnp.int32, (1, S), 1)
    for b in range(B):
        ln = cu_ref[b + 1] - cu_ref[b]
        mask_ref[pl.ds(b, 1), :] = col < ln


_mask_call = pl.pallas_call(
    _mask_body,
    in_specs=[pl.BlockSpec(memory_space=pltpu.SMEM)],
    out_specs=pl.BlockSpec(memory_space=pltpu.VMEM),
    out_shape=jax.ShapeDtypeStruct((B, S), jnp.bool_),
)


def kernel(flat, cu_seqlens):
    cu_p = jnp.pad(cu_seqlens.astype(jnp.int32), (0, 32 - (B + 1)))
    out = _sc_unflatten(flat, cu_p)
    mask = _mask_call(cu_p)
    return out, mask


# speedup vs baseline: 2.8448x; 1.0095x over previous
import functools

import jax
import jax.numpy as jnp
from jax import lax
from jax.experimental import pallas as pl
from jax.experimental.pallas import tpu as pltpu
from jax.experimental.pallas import tpu_sc as plsc

B = 16
S = 2048
T = 16384
D = 512

NW = 32
CH = 32
NPOS = S // CH
NSLOT = 32
NBUF = 6
LAG = 3

_mesh = plsc.VectorSubcoreMesh(core_axis_name="c", subcore_axis_name="s")


@functools.partial(
    pl.kernel,
    mesh=_mesh,
    out_type=jax.ShapeDtypeStruct((B, S, D), jnp.float32),
    scratch_types=[
        pltpu.VMEM((32,), jnp.int32),
        pltpu.VMEM((NBUF, CH), jnp.int32),
        pltpu.VMEM((CH, D), jnp.float32),
        pltpu.VMEM((CH, D), jnp.float32),
        pltpu.VMEM((CH, D), jnp.float32),
        pltpu.VMEM((CH, D), jnp.float32),
        pltpu.VMEM((CH, D), jnp.float32),
        pltpu.VMEM((CH, D), jnp.float32),
        pltpu.VMEM((CH, D), jnp.float32),
        pltpu.SemaphoreType.DMA,
        pltpu.SemaphoreType.DMA,
        pltpu.SemaphoreType.DMA,
        pltpu.SemaphoreType.DMA,
        pltpu.SemaphoreType.DMA,
        pltpu.SemaphoreType.DMA,
        pltpu.SemaphoreType.DMA,
        pltpu.SemaphoreType.DMA,
        pltpu.SemaphoreType.DMA,
        pltpu.SemaphoreType.DMA,
        pltpu.SemaphoreType.DMA,
        pltpu.SemaphoreType.DMA,
        pltpu.SemaphoreType.DMA,
    ],
)
def _sc_unflatten(
    flat_hbm, cu_hbm, out_hbm, cu_v, idx_v,
    buf0, buf1, buf2, buf3, buf4, buf5, zbuf,
    sg0, sg1, sg2, sg3, sg4, sg5,
    sw0, sw1, sw2, sw3, sw4, sw5, sem_z,
):
    buf = [buf0, buf1, buf2, buf3, buf4, buf5]
    sem_g = [sg0, sg1, sg2, sg3, sg4, sg5]
    sem_w = [sw0, sw1, sw2, sw3, sw4, sw5]
    wid = lax.axis_index("c") * 16 + lax.axis_index("s")

    cu_cp = pltpu.async_copy(cu_hbm, cu_v, sg0)

    def _zero(i, carry):
        r = i // 4
        col = (i % 4) * 128
        for u in range(8):
            zbuf[r, pl.ds(col + u * 16, 16)] = jnp.zeros((16,), jnp.float32)
        return carry

    lax.fori_loop(0, CH * D // 128, _zero, 0)
    cu_cp.wait()

    lane = lax.broadcasted_iota(jnp.int32, (16,), 0)

    start_e = []
    end_e = []
    for t in range(B):
        cu_b = cu_v[pl.ds(t, 16)]
        start_e.append(cu_b[0])
        end_e.append(cu_b[1])

    pos = []
    nv_s = []
    base_s = []
    end_s = []
    valid = []
    for s in range(NSLOT):
        t = s % B
        p = (wid + 2 * t + CH * (s // B)) % NPOS
        nv = jnp.clip(end_e[t] - start_e[t] - p * CH, 0, CH)
        pos.append(p)
        nv_s.append(nv)
        base_s.append(start_e[t] + p * CH)
        end_s.append(end_e[t])
        valid.append(nv > 0)

    def _dst(s):
        return out_hbm.at[s % B, pl.ds(pos[s] * CH, CH)]

    for s in range(NSLOT + LAG):
        if s < NSLOT:
            k = s % NBUF
            if s >= NBUF:
                @pl.when(valid[s - NBUF])
                def _():
                    pltpu.make_async_copy(
                        buf[k], _dst(s - NBUF), sem_w[k]
                    ).wait()

            @pl.when(valid[s])
            def _():
                for j in range(CH // 16):
                    idx_v[k, pl.ds(j * 16, 16)] = jnp.minimum(
                        base_s[s] + j * 16 + lane, end_s[s] - 1
                    )
                pltpu.async_copy(
                    flat_hbm.at[idx_v.at[k]], buf[k], sem_g[k]
                )

            @pl.when(jnp.logical_not(valid[s]))
            def _():
                pltpu.async_copy(zbuf, _dst(s), sem_z)

        sp = s - LAG
        if sp >= 0:
            kp = sp % NBUF

            @pl.when(valid[sp])
            def _():
                pltpu.make_async_copy(buf[kp], _dst(sp), sem_g[kp]).wait()
                nv = nv_s[sp]
                def _ztail(i, carry):
                    r = nv + i // 4
                    col = (i % 4) * 128
                    for u in range(8):
                        buf[kp][r, pl.ds(col + u * 16, 16)] = jnp.zeros(
                            (16,), jnp.float32
                        )
                    return carry

                lax.fori_loop(0, (CH - nv) * (D // 128), _ztail, 0)
                pltpu.async_copy(buf[kp], _dst(sp), sem_w[kp])

    for s in range(NSLOT - NBUF, NSLOT):
        @pl.when(valid[s])
        def _():
            pltpu.make_async_copy(
                buf[s % NBUF], _dst(s), sem_w[s % NBUF]
            ).wait()

    n_zero = 0
    for s in range(NSLOT):
        n_zero = n_zero + jnp.where(valid[s], 0, 1)

    def _drain_z(i, carry):
        pltpu.make_async_copy(
            zbuf, out_hbm.at[0, pl.ds(0, CH)], sem_z
        ).wait()
        return carry

    lax.fori_loop(0, n_zero, _drain_z, 0)


def _mask_body(cu_ref, mask_ref):
    col = lax.broadcasted_iota(jnp.int32, (1, S), 1)
    for b in range(B):
        ln = cu_ref[b + 1] - cu_ref[b]
        mask_ref[pl.ds(b, 1), :] = col < ln


_mask_call = pl.pallas_call(
    _mask_body,
    in_specs=[pl.BlockSpec(memory_space=pltpu.SMEM)],
    out_specs=pl.BlockSpec(memory_space=pltpu.VMEM),
    out_shape=jax.ShapeDtypeStruct((B, S), jnp.bool_),
)


def kernel(flat, cu_seqlens):
    cu_p = jnp.pad(cu_seqlens.astype(jnp.int32), (0, 32 - (B + 1)))
    out = _sc_unflatten(flat, cu_p)
    mask = _mask_call(cu_p)
    return out, mask
